# baseline (device time: 132521 ns/iter reference)
import jax
import jax.numpy as jnp
from jax import lax
from jax.experimental import pallas as pl
from jax.experimental.pallas import tpu as pltpu

N_DEV = 16
N_TOK = 2048
D_MODEL = 512
D_HID = 1024
N_EXP = 64
E_LOCAL = N_EXP // N_DEV
CHUNK = N_TOK // N_DEV
N_R = 8
N_L = 7

STRIPS = (
    {"c0": 0, "w": 128},
    {"c0": 128, "w": 128},
    {"c0": 256, "w": 128},
    {"c0": 384, "w": 128},
    {"c0": 512, "w": 128},
    {"c0": 640, "w": 128},
    {"c0": 768, "w": 128},
    {"c0": 896, "w": 128},
)


def kernel(x, router_W, route_idx, expert_W):
    def body(x_ref, rw_ref, idx_ref, ew_ref, out_ref, acc_ref, w_ref, *sc):
        my = lax.axis_index("i")

        def mod(v):
            return lax.rem(v + 4 * N_DEV, N_DEV)

        right = mod(my + 1)
        left = mod(my - 1)

        def sc_of(si):
            return sc[si * 10 : (si + 1) * 10]

        def rows_of(c):
            return pl.ds(c * CHUNK, CHUNK)

        all_descs = []
        rsR = [[None] * N_R for _ in STRIPS]
        rsL = [[None] * N_L for _ in STRIPS]
        agR = [[None] * N_R for _ in STRIPS]
        agL = [[None] * N_L for _ in STRIPS]

        def start_rsR(si, s):
            st = STRIPS[si]
            commR = sc_of(si)[0]
            d = pltpu.make_async_remote_copy(
                src_ref=acc_ref.at[rows_of(mod(my + 8 - s)), pl.ds(st["c0"], st["w"])],
                dst_ref=commR.at[s],
                send_sem=sc_of(si)[2].at[s],
                recv_sem=sc_of(si)[3].at[s],
                device_id=(right,),
                device_id_type=pl.DeviceIdType.MESH,
            )
            d.start()
            all_descs.append(d)
            rsR[si][s] = d

        def start_rsL(si, s):
            st = STRIPS[si]
            commL = sc_of(si)[1]
            d = pltpu.make_async_remote_copy(
                src_ref=acc_ref.at[rows_of(mod(my - 7 + s)), pl.ds(st["c0"], st["w"])],
                dst_ref=commL.at[s],
                send_sem=sc_of(si)[4].at[s],
                recv_sem=sc_of(si)[5].at[s],
                device_id=(left,),
                device_id_type=pl.DeviceIdType.MESH,
            )
            d.start()
            all_descs.append(d)
            rsL[si][s] = d

        def start_agR(si, h):
            st = STRIPS[si]
            blk = out_ref.at[rows_of(mod(my - h)), pl.ds(st["c0"], st["w"])]
            d = pltpu.make_async_remote_copy(
                src_ref=blk,
                dst_ref=blk,
                send_sem=sc_of(si)[6].at[h],
                recv_sem=sc_of(si)[7].at[h],
                device_id=(right,),
                device_id_type=pl.DeviceIdType.MESH,
            )
            d.start()
            all_descs.append(d)
            agR[si][h] = d

        def start_agL(si, h):
            st = STRIPS[si]
            blk = out_ref.at[rows_of(mod(my + h)), pl.ds(st["c0"], st["w"])]
            d = pltpu.make_async_remote_copy(
                src_ref=blk,
                dst_ref=blk,
                send_sem=sc_of(si)[8].at[h],
                recv_sem=sc_of(si)[9].at[h],
                device_id=(left,),
                device_id_type=pl.DeviceIdType.MESH,
            )
            d.start()
            all_descs.append(d)
            agL[si][h] = d

        xv = x_ref[...]
        scores = jnp.dot(xv, rw_ref[...], preferred_element_type=jnp.float32)
        m = jnp.max(scores, axis=-1, keepdims=True)
        p = jnp.exp(scores - m)
        p = p / jnp.sum(p, axis=-1, keepdims=True)
        idx = idx_ref[...]
        e0 = idx[:, 0:1]
        e1 = idx[:, 1:2]
        cols = lax.broadcasted_iota(jnp.int32, (N_TOK, N_EXP), 1)
        g0 = jnp.sum(jnp.where(cols == e0, p, 0.0), axis=-1, keepdims=True)
        g1 = jnp.sum(jnp.where(cols == e1, p, 0.0), axis=-1, keepdims=True)
        gs = g0 + g1
        for le in range(E_LOCAL):
            ge = my * E_LOCAL + le
            w = jnp.where(e0 == ge, g0 / gs, 0.0) + jnp.where(e1 == ge, g1 / gs, 0.0)
            w_ref[:, le : le + 1] = w
        ew_flat = ew_ref[...].reshape(E_LOCAL * D_MODEL, D_HID)

        def compute_chunk(c):
            xc = x_ref[pl.ds(c * CHUNK, CHUNK), :]
            xcw = jnp.concatenate(
                [
                    xc * w_ref[pl.ds(c * CHUNK, CHUNK), le : le + 1]
                    for le in range(E_LOCAL)
                ],
                axis=1,
            )
            acc_ref[pl.ds(c * CHUNK, CHUNK), :] = jnp.dot(
                xcw, ew_flat, preferred_element_type=jnp.float32
            )

        compute_chunk(mod(my + 8))
        compute_chunk(mod(my + 9))
        for si in range(len(STRIPS)):
            start_rsR(si, 0)
            start_rsL(si, 0)
        for r in range(N_R):
            if r < N_R - 1:
                compute_chunk(mod(my + 7 - r))
                compute_chunk(mod(my + 10 + r))
            for si, st in enumerate(STRIPS):
                cols_sl = pl.ds(st["c0"], st["w"])
                commR = sc_of(si)[0]
                rsR[si][r].wait_recv()
                rR = rows_of(mod(my + 7 - r))
                acc_ref[rR, cols_sl] = acc_ref[rR, cols_sl] + commR[r]
                if r + 1 < N_R:
                    start_rsR(si, r + 1)
                if r < N_L:
                    commL = sc_of(si)[1]
                    rsL[si][r].wait_recv()
                    rL = rows_of(mod(my - 6 + r))
                    acc_ref[rL, cols_sl] = acc_ref[rL, cols_sl] + commL[r]
                    if r + 1 < N_L:
                        start_rsL(si, r + 1)
                if r == N_R - 1:
                    own_rows = rows_of(my)
                    out_ref[own_rows, cols_sl] = acc_ref[own_rows, cols_sl]
                    start_agR(si, 0)
                    start_agL(si, 0)

        for r in range(N_R):
            for si in range(len(STRIPS)):
                agR[si][r].wait_recv()
                if r + 1 < N_R:
                    start_agR(si, r + 1)
                if r < N_L:
                    agL[si][r].wait_recv()
                    if r + 1 < N_L:
                        start_agL(si, r + 1)

        for d in all_descs:
            d.wait_send()

    scratch = [
        pltpu.VMEM((N_TOK, D_HID), jnp.float32),
        pltpu.VMEM((N_TOK, E_LOCAL), jnp.float32),
    ]
    for st in STRIPS:
        scratch += [
            pltpu.VMEM((N_R, CHUNK, st["w"]), jnp.float32),
            pltpu.VMEM((N_L, CHUNK, st["w"]), jnp.float32),
            pltpu.SemaphoreType.DMA((N_R,)),
            pltpu.SemaphoreType.DMA((N_R,)),
            pltpu.SemaphoreType.DMA((N_L,)),
            pltpu.SemaphoreType.DMA((N_L,)),
            pltpu.SemaphoreType.DMA((N_R,)),
            pltpu.SemaphoreType.DMA((N_R,)),
            pltpu.SemaphoreType.DMA((N_L,)),
            pltpu.SemaphoreType.DMA((N_L,)),
        ]

    return pl.pallas_call(
        body,
        out_shape=jax.ShapeDtypeStruct((N_TOK, D_HID), jnp.float32),
        in_specs=[pl.BlockSpec(memory_space=pltpu.VMEM)] * 4,
        out_specs=pl.BlockSpec(memory_space=pltpu.VMEM),
        scratch_shapes=scratch,
        compiler_params=pltpu.CompilerParams(
            vmem_limit_bytes=100 * 1024 * 1024,
        ),
    )(x, router_W, route_idx, expert_W)


# device time: 130177 ns/iter; 1.0180x vs baseline; 1.0180x over previous
import jax
import jax.numpy as jnp
from jax import lax
from jax.experimental import pallas as pl
from jax.experimental.pallas import tpu as pltpu

N_DEV = 16
N_TOK = 2048
D_MODEL = 512
D_HID = 1024
N_EXP = 64
E_LOCAL = N_EXP // N_DEV
CHUNK = N_TOK // N_DEV
N_R = 8
N_L = 7

STRIPS = (
    {"c0": 0, "w": 256},
    {"c0": 256, "w": 256},
    {"c0": 512, "w": 256},
    {"c0": 768, "w": 256},
)


def kernel(x, router_W, route_idx, expert_W):
    def body(x_ref, rw_ref, idx_ref, ew_ref, out_ref, acc_ref, w_ref, *sc):
        my = lax.axis_index("i")

        def mod(v):
            return lax.rem(v + 4 * N_DEV, N_DEV)

        right = mod(my + 1)
        left = mod(my - 1)

        def sc_of(si):
            return sc[si * 10 : (si + 1) * 10]

        def rows_of(c):
            return pl.ds(c * CHUNK, CHUNK)

        all_descs = []
        rsR = [[None] * N_R for _ in STRIPS]
        rsL = [[None] * N_L for _ in STRIPS]
        agR = [[None] * N_R for _ in STRIPS]
        agL = [[None] * N_L for _ in STRIPS]

        def start_rsR(si, s):
            st = STRIPS[si]
            commR = sc_of(si)[0]
            d = pltpu.make_async_remote_copy(
                src_ref=acc_ref.at[rows_of(mod(my + 8 - s)), pl.ds(st["c0"], st["w"])],
                dst_ref=commR.at[s],
                send_sem=sc_of(si)[2].at[s],
                recv_sem=sc_of(si)[3].at[s],
                device_id=(right,),
                device_id_type=pl.DeviceIdType.MESH,
            )
            d.start()
            all_descs.append(d)
            rsR[si][s] = d

        def start_rsL(si, s):
            st = STRIPS[si]
            commL = sc_of(si)[1]
            d = pltpu.make_async_remote_copy(
                src_ref=acc_ref.at[rows_of(mod(my - 7 + s)), pl.ds(st["c0"], st["w"])],
                dst_ref=commL.at[s],
                send_sem=sc_of(si)[4].at[s],
                recv_sem=sc_of(si)[5].at[s],
                device_id=(left,),
                device_id_type=pl.DeviceIdType.MESH,
            )
            d.start()
            all_descs.append(d)
            rsL[si][s] = d

        def start_agR(si, h):
            st = STRIPS[si]
            blk = out_ref.at[rows_of(mod(my - h)), pl.ds(st["c0"], st["w"])]
            d = pltpu.make_async_remote_copy(
                src_ref=blk,
                dst_ref=blk,
                send_sem=sc_of(si)[6].at[h],
                recv_sem=sc_of(si)[7].at[h],
                device_id=(right,),
                device_id_type=pl.DeviceIdType.MESH,
            )
            d.start()
            all_descs.append(d)
            agR[si][h] = d

        def start_agL(si, h):
            st = STRIPS[si]
            blk = out_ref.at[rows_of(mod(my + h)), pl.ds(st["c0"], st["w"])]
            d = pltpu.make_async_remote_copy(
                src_ref=blk,
                dst_ref=blk,
                send_sem=sc_of(si)[8].at[h],
                recv_sem=sc_of(si)[9].at[h],
                device_id=(left,),
                device_id_type=pl.DeviceIdType.MESH,
            )
            d.start()
            all_descs.append(d)
            agL[si][h] = d

        xv = x_ref[...]
        scores = jnp.dot(xv, rw_ref[...], preferred_element_type=jnp.float32)
        m = jnp.max(scores, axis=-1, keepdims=True)
        p = jnp.exp(scores - m)
        p = p / jnp.sum(p, axis=-1, keepdims=True)
        idx = idx_ref[...]
        e0 = idx[:, 0:1]
        e1 = idx[:, 1:2]
        cols = lax.broadcasted_iota(jnp.int32, (N_TOK, N_EXP), 1)
        g0 = jnp.sum(jnp.where(cols == e0, p, 0.0), axis=-1, keepdims=True)
        g1 = jnp.sum(jnp.where(cols == e1, p, 0.0), axis=-1, keepdims=True)
        gs = g0 + g1
        for le in range(E_LOCAL):
            ge = my * E_LOCAL + le
            w = jnp.where(e0 == ge, g0 / gs, 0.0) + jnp.where(e1 == ge, g1 / gs, 0.0)
            w_ref[:, le : le + 1] = w
        ew_flat = ew_ref[...].reshape(E_LOCAL * D_MODEL, D_HID)

        def compute_chunk(c):
            xc = x_ref[pl.ds(c * CHUNK, CHUNK), :]
            xcw = jnp.concatenate(
                [
                    xc * w_ref[pl.ds(c * CHUNK, CHUNK), le : le + 1]
                    for le in range(E_LOCAL)
                ],
                axis=1,
            )
            acc_ref[pl.ds(c * CHUNK, CHUNK), :] = jnp.dot(
                xcw, ew_flat, preferred_element_type=jnp.float32
            )

        compute_chunk(mod(my + 8))
        compute_chunk(mod(my + 9))
        for si in range(len(STRIPS)):
            start_rsR(si, 0)
            start_rsL(si, 0)
        for r in range(N_R):
            if r < N_R - 1:
                compute_chunk(mod(my + 7 - r))
                compute_chunk(mod(my + 10 + r))
            for si, st in enumerate(STRIPS):
                cols_sl = pl.ds(st["c0"], st["w"])
                commR = sc_of(si)[0]
                rsR[si][r].wait_recv()
                rR = rows_of(mod(my + 7 - r))
                acc_ref[rR, cols_sl] = acc_ref[rR, cols_sl] + commR[r]
                if r + 1 < N_R:
                    start_rsR(si, r + 1)
                if r < N_L:
                    commL = sc_of(si)[1]
                    rsL[si][r].wait_recv()
                    rL = rows_of(mod(my - 6 + r))
                    acc_ref[rL, cols_sl] = acc_ref[rL, cols_sl] + commL[r]
                    if r + 1 < N_L:
                        start_rsL(si, r + 1)
                if r == N_R - 1:
                    own_rows = rows_of(my)
                    out_ref[own_rows, cols_sl] = acc_ref[own_rows, cols_sl]
                    start_agR(si, 0)
                    start_agL(si, 0)

        for r in range(N_R):
            for si in range(len(STRIPS)):
                agR[si][r].wait_recv()
                if r + 1 < N_R:
                    start_agR(si, r + 1)
                if r < N_L:
                    agL[si][r].wait_recv()
                    if r + 1 < N_L:
                        start_agL(si, r + 1)

        for d in all_descs:
            d.wait_send()

    scratch = [
        pltpu.VMEM((N_TOK, D_HID), jnp.float32),
        pltpu.VMEM((N_TOK, E_LOCAL), jnp.float32),
    ]
    for st in STRIPS:
        scratch += [
            pltpu.VMEM((N_R, CHUNK, st["w"]), jnp.float32),
            pltpu.VMEM((N_L, CHUNK, st["w"]), jnp.float32),
            pltpu.SemaphoreType.DMA((N_R,)),
            pltpu.SemaphoreType.DMA((N_R,)),
            pltpu.SemaphoreType.DMA((N_L,)),
            pltpu.SemaphoreType.DMA((N_L,)),
            pltpu.SemaphoreType.DMA((N_R,)),
            pltpu.SemaphoreType.DMA((N_R,)),
            pltpu.SemaphoreType.DMA((N_L,)),
            pltpu.SemaphoreType.DMA((N_L,)),
        ]

    return pl.pallas_call(
        body,
        out_shape=jax.ShapeDtypeStruct((N_TOK, D_HID), jnp.float32),
        in_specs=[pl.BlockSpec(memory_space=pltpu.VMEM)] * 4,
        out_specs=pl.BlockSpec(memory_space=pltpu.VMEM),
        scratch_shapes=scratch,
        compiler_params=pltpu.CompilerParams(
            vmem_limit_bytes=100 * 1024 * 1024,
        ),
    )(x, router_W, route_idx, expert_W)
